# baseline (speedup 1.0000x reference)
import functools

import jax
import jax.numpy as jnp
from jax import lax
from jax.experimental import pallas as pl
from jax.experimental.pallas import tpu as pltpu
from jax.experimental.pallas import tpu_sc as plsc

_B, _N, _C = 8, 115200, 3
_ROWS = _C * _B
_COLS = _N
_NC, _NS = 2, 16
_NW = _NC * _NS
_CW = 1152
_CPR = _COLS // _CW
_NCHUNK = (_ROWS // 8) * _CPR
_K_MAX = (_NCHUNK + _NW - 1) // _NW

_mesh = plsc.VectorSubcoreMesh(core_axis_name="c", subcore_axis_name="s")


@functools.partial(
    pl.kernel,
    mesh=_mesh,
    out_type=jax.ShapeDtypeStruct((_ROWS, _COLS), jnp.float32),
    scratch_types=[
        pltpu.VMEM((8, _CW), jnp.float32),
        pltpu.VMEM((8, _CW), jnp.float32),
        pltpu.VMEM((8, _CW), jnp.float32),
        pltpu.VMEM((8, _CW), jnp.float32),
        pltpu.VMEM((8, _CW), jnp.float32),
        pltpu.VMEM((8, _CW), jnp.float32),
        pltpu.SemaphoreType.DMA,
        pltpu.SemaphoreType.DMA,
        pltpu.SemaphoreType.DMA,
        pltpu.SemaphoreType.DMA,
    ],
)
def _sc_add(x_hbm, f_hbm, o_hbm, a0, b0, a1, b1, o0, o1, si0, si1, so0, so1):
    wid = lax.axis_index("s") * _NC + lax.axis_index("c")
    av = (a0, a1)
    bv = (b0, b1)
    ov = (o0, o1)
    si = (si0, si1)
    so = (so0, so1)

    def src_slice(k):
        c = wid + k * _NW
        return (pl.ds((c // _CPR) * 8, 8), pl.ds((c % _CPR) * _CW, _CW))

    def start_in(k, p):
        @pl.when(wid + k * _NW < _NCHUNK)
        def _():
            s = src_slice(k)
            pltpu.async_copy(x_hbm.at[s], av[p], si[p])
            pltpu.async_copy(f_hbm.at[s], bv[p], si[p])

    start_in(0, 0)
    start_in(1, 1)

    @pl.loop(0, _K_MAX, step=2)
    def _(kk):
        for p in range(2):
            k = kk + p

            @pl.when(wid + k * _NW < _NCHUNK)
            def _():
                s = src_slice(k)
                pltpu.make_async_copy(x_hbm.at[s], av[p], si[p]).wait()
                pltpu.make_async_copy(f_hbm.at[s], bv[p], si[p]).wait()

                @pl.when(k >= 2)
                def _():
                    pltpu.make_async_copy(ov[p], o_hbm.at[s], so[p]).wait()

                def quarter(q, c2):
                    r = q // 4
                    base = (q % 4) * (_CW // 4)
                    for cc in range(_CW // 64):
                        sl = pl.ds(base + cc * 16, 16)
                        ov[p][r, sl] = av[p][r, sl] + bv[p][r, sl]
                    return c2

                lax.fori_loop(0, 32, quarter, 0)
                pltpu.async_copy(ov[p], o_hbm.at[s], so[p])
                start_in(k + 2, p)

    drain = (pl.ds(0, 8), pl.ds(0, _CW))
    for p in range(2):
        pltpu.make_async_copy(ov[p], o_hbm.at[drain], so[p]).wait()


def kernel(xyz1, upsampled_flow):
    x = jnp.transpose(xyz1, (2, 0, 1)).reshape(_ROWS, _COLS)
    f = jnp.transpose(upsampled_flow, (2, 0, 1)).reshape(_ROWS, _COLS)
    out = _sc_add(x, f)
    return jnp.transpose(out.reshape(_C, _B, _N), (1, 2, 0))


# speedup vs baseline: 1.1712x; 1.1712x over previous
import functools

import jax
import jax.numpy as jnp
from jax import lax
from jax.experimental import pallas as pl
from jax.experimental.pallas import tpu as pltpu
from jax.experimental.pallas import tpu_sc as plsc

_B, _N, _C = 8, 115200, 3
_ROWS = _C * _B
_COLS = _N
_NC, _NS = 2, 16
_NW = _NC * _NS
_CW = 512
_NCHUNK = _COLS // _CW
_K_MAX = (_NCHUNK + _NW - 1) // _NW

_mesh = plsc.VectorSubcoreMesh(core_axis_name="c", subcore_axis_name="s")


@functools.partial(
    pl.kernel,
    mesh=_mesh,
    out_type=jax.ShapeDtypeStruct((_ROWS, _COLS), jnp.float32),
    scratch_types=[
        pltpu.VMEM((8, _CW), jnp.float32),
        pltpu.VMEM((8, _CW), jnp.float32),
        pltpu.VMEM((8, _CW), jnp.float32),
        pltpu.VMEM((8, _CW), jnp.float32),
        pltpu.VMEM((8, _CW), jnp.float32),
        pltpu.VMEM((8, _CW), jnp.float32),
        pltpu.SemaphoreType.DMA,
        pltpu.SemaphoreType.DMA,
        pltpu.SemaphoreType.DMA,
        pltpu.SemaphoreType.DMA,
    ],
)
def _sc_add_band0(x_hbm, f_hbm, o_hbm, a0, b0, a1, b1, o0, o1, si0, si1, so0, so1):
    wid = lax.axis_index("s") * _NC + lax.axis_index("c")
    av = (a0, a1)
    bv = (b0, b1)
    ov = (o0, o1)
    si = (si0, si1)
    so = (so0, so1)

    def src_slice(k):
        c = wid + k * _NW
        return (pl.ds(0, 8), pl.ds(c * _CW, _CW))

    def start_in(k, p):
        @pl.when(wid + k * _NW < _NCHUNK)
        def _():
            s = src_slice(k)
            pltpu.async_copy(x_hbm.at[s], av[p], si[p])
            pltpu.async_copy(f_hbm.at[s], bv[p], si[p])

    start_in(0, 0)
    start_in(1, 1)

    @pl.loop(0, _K_MAX, step=2)
    def _(kk):
        for p in range(2):
            k = kk + p

            @pl.when(wid + k * _NW < _NCHUNK)
            def _():
                s = src_slice(k)
                pltpu.make_async_copy(x_hbm.at[s], av[p], si[p]).wait()
                pltpu.make_async_copy(f_hbm.at[s], bv[p], si[p]).wait()

                @pl.when(k >= 2)
                def _():
                    pltpu.make_async_copy(ov[p], o_hbm.at[s], so[p]).wait()

                def row(r, c2):
                    for cc in range(_CW // 16):
                        sl = pl.ds(cc * 16, 16)
                        ov[p][r, sl] = av[p][r, sl] + bv[p][r, sl]
                    return c2

                lax.fori_loop(0, 8, row, 0)
                pltpu.async_copy(ov[p], o_hbm.at[s], so[p])
                start_in(k + 2, p)

    drain = (pl.ds(0, 8), pl.ds(0, _CW))
    for p in range(2):

        @pl.when(wid + p * _NW < _NCHUNK)
        def _():
            pltpu.make_async_copy(ov[p], o_hbm.at[drain], so[p]).wait()


_TCW = 11520
_TC_NB = _COLS // _TCW


def _tc_body(x_ref, f_ref, carrier_ref, o_ref):
    o_ref[...] = x_ref[...] + f_ref[...]


_tc_add_bands12 = pl.pallas_call(
    _tc_body,
    grid=(2, _TC_NB),
    in_specs=[
        pl.BlockSpec((8, _TCW), lambda r, i: (r + 1, i)),
        pl.BlockSpec((8, _TCW), lambda r, i: (r + 1, i)),
        pl.BlockSpec((8, _TCW), lambda r, i: (r + 1, i)),
    ],
    out_specs=pl.BlockSpec((8, _TCW), lambda r, i: (r + 1, i)),
    out_shape=jax.ShapeDtypeStruct((_ROWS, _COLS), jnp.float32),
    input_output_aliases={2: 0},
)


def kernel(xyz1, upsampled_flow):
    x = jnp.transpose(xyz1, (2, 0, 1)).reshape(_ROWS, _COLS)
    f = jnp.transpose(upsampled_flow, (2, 0, 1)).reshape(_ROWS, _COLS)
    partial = _sc_add_band0(x, f)
    out = _tc_add_bands12(x, f, partial)
    return jnp.transpose(out.reshape(_C, _B, _N), (1, 2, 0))


# speedup vs baseline: 1.3156x; 1.1233x over previous
import functools

import jax
import jax.numpy as jnp
from jax import lax
from jax.experimental import pallas as pl
from jax.experimental.pallas import tpu as pltpu
from jax.experimental.pallas import tpu_sc as plsc

_B, _N, _C = 8, 115200, 3
_ROWS = _C * _B
_COLS = _N
_NC, _NS = 2, 16
_NW = _NC * _NS
_CW = 512
_NCHUNK = _COLS // _CW
_K_MAX = (_NCHUNK + _NW - 1) // _NW

_mesh = plsc.VectorSubcoreMesh(core_axis_name="c", subcore_axis_name="s")


@functools.partial(
    pl.kernel,
    mesh=_mesh,
    out_type=jax.ShapeDtypeStruct((_ROWS, _COLS), jnp.float32),
    scratch_types=[
        pltpu.VMEM((8, _CW), jnp.float32),
        pltpu.VMEM((8, _CW), jnp.float32),
        pltpu.VMEM((8, _CW), jnp.float32),
        pltpu.VMEM((8, _CW), jnp.float32),
        pltpu.VMEM((8, _CW), jnp.float32),
        pltpu.VMEM((8, _CW), jnp.float32),
        pltpu.SemaphoreType.DMA,
        pltpu.SemaphoreType.DMA,
        pltpu.SemaphoreType.DMA,
        pltpu.SemaphoreType.DMA,
    ],
)
def _sc_add_band0(x_hbm, f_hbm, o_hbm, a0, b0, a1, b1, o0, o1, si0, si1, so0, so1):
    wid = lax.axis_index("s") * _NC + lax.axis_index("c")
    av = (a0, a1)
    bv = (b0, b1)
    ov = (o0, o1)
    si = (si0, si1)
    so = (so0, so1)

    def src_slice(k):
        c = wid + k * _NW
        return (pl.ds(16, 8), pl.ds(c * _CW, _CW))

    def start_in(k, p):
        @pl.when(wid + k * _NW < _NCHUNK)
        def _():
            s = src_slice(k)
            pltpu.async_copy(x_hbm.at[s], av[p], si[p])
            pltpu.async_copy(f_hbm.at[s], bv[p], si[p])

    start_in(0, 0)
    start_in(1, 1)

    @pl.loop(0, _K_MAX, step=2)
    def _(kk):
        for p in range(2):
            k = kk + p

            @pl.when(wid + k * _NW < _NCHUNK)
            def _():
                s = src_slice(k)
                pltpu.make_async_copy(x_hbm.at[s], av[p], si[p]).wait()
                pltpu.make_async_copy(f_hbm.at[s], bv[p], si[p]).wait()

                @pl.when(k >= 2)
                def _():
                    pltpu.make_async_copy(ov[p], o_hbm.at[s], so[p]).wait()

                def row(r, c2):
                    for cc in range(_CW // 16):
                        sl = pl.ds(cc * 16, 16)
                        ov[p][r, sl] = av[p][r, sl] + bv[p][r, sl]
                    return c2

                lax.fori_loop(0, 8, row, 0)
                pltpu.async_copy(ov[p], o_hbm.at[s], so[p])
                start_in(k + 2, p)

    drain = (pl.ds(16, 8), pl.ds(0, _CW))
    for p in range(2):

        @pl.when(wid + p * _NW < _NCHUNK)
        def _():
            pltpu.make_async_copy(ov[p], o_hbm.at[drain], so[p]).wait()


_TCW = 11520
_TC_NB = _COLS // _TCW


def _tc_body(x_ref, f_ref, carrier_ref, o_ref):
    o_ref[...] = x_ref[...] + f_ref[...]


_tc_add_bands01 = pl.pallas_call(
    _tc_body,
    grid=(_TC_NB,),
    in_specs=[
        pl.BlockSpec((16, _TCW), lambda i: (0, i)),
        pl.BlockSpec((16, _TCW), lambda i: (0, i)),
        pl.BlockSpec((16, _TCW), lambda i: (0, i)),
    ],
    out_specs=pl.BlockSpec((16, _TCW), lambda i: (0, i)),
    out_shape=jax.ShapeDtypeStruct((_ROWS, _COLS), jnp.float32),
    input_output_aliases={2: 0},
)


def kernel(xyz1, upsampled_flow):
    x = jnp.transpose(xyz1, (2, 0, 1)).reshape(_ROWS, _COLS)
    f = jnp.transpose(upsampled_flow, (2, 0, 1)).reshape(_ROWS, _COLS)
    partial = _sc_add_band0(x, f)
    out = _tc_add_bands01(x, f, partial)
    return jnp.transpose(out.reshape(_C, _B, _N), (1, 2, 0))


# speedup vs baseline: 1.4318x; 1.0884x over previous
import functools

import jax
import jax.numpy as jnp
from jax import lax
from jax.experimental import pallas as pl
from jax.experimental.pallas import tpu as pltpu
from jax.experimental.pallas import tpu_sc as plsc

_B, _N, _C = 8, 115200, 3
_ROWS = _C * _B
_COLS = _N
_NC, _NS = 2, 16
_NW = _NC * _NS
_CW = 512
_NCHUNK = _COLS // _CW
_K_MAX = (_NCHUNK + _NW - 1) // _NW

_mesh = plsc.VectorSubcoreMesh(core_axis_name="c", subcore_axis_name="s")


@functools.partial(
    pl.kernel,
    mesh=_mesh,
    out_type=jax.ShapeDtypeStruct((_ROWS, _COLS), jnp.float32),
    scratch_types=[
        pltpu.VMEM((8, _CW), jnp.float32),
        pltpu.VMEM((8, _CW), jnp.float32),
        pltpu.VMEM((8, _CW), jnp.float32),
        pltpu.VMEM((8, _CW), jnp.float32),
        pltpu.VMEM((8, _CW), jnp.float32),
        pltpu.VMEM((8, _CW), jnp.float32),
        pltpu.SemaphoreType.DMA,
        pltpu.SemaphoreType.DMA,
        pltpu.SemaphoreType.DMA,
        pltpu.SemaphoreType.DMA,
    ],
)
def _sc_add_band0(x_hbm, f_hbm, o_hbm, a0, b0, a1, b1, o0, o1, si0, si1, so0, so1):
    wid = lax.axis_index("s") * _NC + lax.axis_index("c")
    av = (a0, a1)
    bv = (b0, b1)
    ov = (o0, o1)
    si = (si0, si1)
    so = (so0, so1)

    def src_slice(k):
        c = wid + k * _NW
        return (pl.ds(16, 8), pl.ds(c * _CW, _CW))

    def start_in(k, p):
        @pl.when(wid + k * _NW < _NCHUNK)
        def _():
            s = src_slice(k)
            pltpu.async_copy(x_hbm.at[s], av[p], si[p])
            pltpu.async_copy(f_hbm.at[s], bv[p], si[p])

    start_in(0, 0)
    start_in(1, 1)

    @pl.loop(0, _K_MAX, step=2)
    def _(kk):
        for p in range(2):
            k = kk + p

            @pl.when(wid + k * _NW < _NCHUNK)
            def _():
                s = src_slice(k)
                pltpu.make_async_copy(x_hbm.at[s], av[p], si[p]).wait()
                pltpu.make_async_copy(f_hbm.at[s], bv[p], si[p]).wait()

                @pl.when(k >= 2)
                def _():
                    pltpu.make_async_copy(ov[p], o_hbm.at[s], so[p]).wait()

                def row(r, c2):
                    for cc in range(_CW // 16):
                        sl = pl.ds(cc * 16, 16)
                        ov[p][r, sl] = av[p][r, sl] + bv[p][r, sl]
                    return c2

                lax.fori_loop(0, 8, row, 0)
                pltpu.async_copy(ov[p], o_hbm.at[s], so[p])
                start_in(k + 2, p)

    drain = (pl.ds(16, 8), pl.ds(0, _CW))
    for p in range(2):

        @pl.when(wid + p * _NW < _NCHUNK)
        def _():
            pltpu.make_async_copy(ov[p], o_hbm.at[drain], so[p]).wait()


_TCW = 23040
_TC_NB = _COLS // _TCW


def _tc_body(x_ref, f_ref, carrier_ref, o_ref):
    o_ref[...] = x_ref[...] + f_ref[...]


_tc_add_bands01 = pl.pallas_call(
    _tc_body,
    grid=(_TC_NB,),
    in_specs=[
        pl.BlockSpec((16, _TCW), lambda i: (0, i)),
        pl.BlockSpec((16, _TCW), lambda i: (0, i)),
        pl.BlockSpec(memory_space=pl.ANY),
    ],
    out_specs=pl.BlockSpec((16, _TCW), lambda i: (0, i)),
    out_shape=jax.ShapeDtypeStruct((_ROWS, _COLS), jnp.float32),
    input_output_aliases={2: 0},
)


def kernel(xyz1, upsampled_flow):
    x = jnp.transpose(xyz1, (2, 0, 1)).reshape(_ROWS, _COLS)
    f = jnp.transpose(upsampled_flow, (2, 0, 1)).reshape(_ROWS, _COLS)
    partial = _sc_add_band0(x, f)
    out = _tc_add_bands01(x, f, partial)
    return jnp.transpose(out.reshape(_C, _B, _N), (1, 2, 0))


# speedup vs baseline: 1.4321x; 1.0002x over previous
import functools

import jax
import jax.numpy as jnp
from jax import lax
from jax.experimental import pallas as pl
from jax.experimental.pallas import tpu as pltpu
from jax.experimental.pallas import tpu_sc as plsc

_B, _N, _C = 8, 115200, 3
_ROWS = _C * _B
_COLS = _N
_NC, _NS = 2, 16
_NW = _NC * _NS
_CW = 768
_NCHUNK = _COLS // _CW
_K_MAX = (_NCHUNK + _NW - 1) // _NW

_mesh = plsc.VectorSubcoreMesh(core_axis_name="c", subcore_axis_name="s")


@functools.partial(
    pl.kernel,
    mesh=_mesh,
    out_type=jax.ShapeDtypeStruct((_ROWS, _COLS), jnp.float32),
    scratch_types=[
        pltpu.VMEM((8, _CW), jnp.float32),
        pltpu.VMEM((8, _CW), jnp.float32),
        pltpu.VMEM((8, _CW), jnp.float32),
        pltpu.VMEM((8, _CW), jnp.float32),
        pltpu.VMEM((8, _CW), jnp.float32),
        pltpu.VMEM((8, _CW), jnp.float32),
        pltpu.SemaphoreType.DMA,
        pltpu.SemaphoreType.DMA,
        pltpu.SemaphoreType.DMA,
        pltpu.SemaphoreType.DMA,
    ],
)
def _sc_add_band0(x_hbm, f_hbm, o_hbm, a0, b0, a1, b1, o0, o1, si0, si1, so0, so1):
    wid = lax.axis_index("s") * _NC + lax.axis_index("c")
    av = (a0, a1)
    bv = (b0, b1)
    ov = (o0, o1)
    si = (si0, si1)
    so = (so0, so1)

    def src_slice(k):
        c = wid + k * _NW
        return (pl.ds(16, 8), pl.ds(c * _CW, _CW))

    def start_in(k, p):
        @pl.when(wid + k * _NW < _NCHUNK)
        def _():
            s = src_slice(k)
            pltpu.async_copy(x_hbm.at[s], av[p], si[p])
            pltpu.async_copy(f_hbm.at[s], bv[p], si[p])

    start_in(0, 0)
    start_in(1, 1)

    @pl.loop(0, _K_MAX, step=2)
    def _(kk):
        for p in range(2):
            k = kk + p

            @pl.when(wid + k * _NW < _NCHUNK)
            def _():
                s = src_slice(k)
                pltpu.make_async_copy(x_hbm.at[s], av[p], si[p]).wait()
                pltpu.make_async_copy(f_hbm.at[s], bv[p], si[p]).wait()

                @pl.when(k >= 2)
                def _():
                    pltpu.make_async_copy(ov[p], o_hbm.at[s], so[p]).wait()

                def row(r, c2):
                    for cc in range(_CW // 16):
                        sl = pl.ds(cc * 16, 16)
                        ov[p][r, sl] = av[p][r, sl] + bv[p][r, sl]
                    return c2

                lax.fori_loop(0, 8, row, 0)
                pltpu.async_copy(ov[p], o_hbm.at[s], so[p])
                start_in(k + 2, p)

    drain = (pl.ds(16, 8), pl.ds(0, _CW))
    for p in range(2):

        @pl.when(wid + p * _NW < _NCHUNK)
        def _():
            pltpu.make_async_copy(ov[p], o_hbm.at[drain], so[p]).wait()


_TCW = 38400
_TC_NB = _COLS // _TCW


def _tc_body(x_ref, f_ref, carrier_ref, o_ref):
    o_ref[...] = x_ref[...] + f_ref[...]


_tc_add_bands01 = pl.pallas_call(
    _tc_body,
    grid=(_TC_NB,),
    in_specs=[
        pl.BlockSpec((16, _TCW), lambda i: (0, i)),
        pl.BlockSpec((16, _TCW), lambda i: (0, i)),
        pl.BlockSpec(memory_space=pl.ANY),
    ],
    out_specs=pl.BlockSpec((16, _TCW), lambda i: (0, i)),
    out_shape=jax.ShapeDtypeStruct((_ROWS, _COLS), jnp.float32),
    input_output_aliases={2: 0},
)


def kernel(xyz1, upsampled_flow):
    x = jnp.transpose(xyz1, (2, 0, 1)).reshape(_ROWS, _COLS)
    f = jnp.transpose(upsampled_flow, (2, 0, 1)).reshape(_ROWS, _COLS)
    partial = _sc_add_band0(x, f)
    out = _tc_add_bands01(x, f, partial)
    return jnp.transpose(out.reshape(_C, _B, _N), (1, 2, 0))
